# baseline (speedup 1.0000x reference)
import functools

import jax
import jax.numpy as jnp
from jax import lax
from jax.experimental import pallas as pl
from jax.experimental.pallas import tpu as pltpu
from jax.experimental.pallas import tpu_sc as plsc

B = 1024
NUM_TYPES = 100000
TYPE_EMB_DIM = 64
PRODUCT_EMB_DIM = 128
FEATURE_DIM = 128
HIDDEN_DIM = 128
K = 8
NUM_NEIGHBORS = 10

T_BLK = 2048
NT = (NUM_TYPES + T_BLK - 1) // T_BLK
NS_SLICES = T_BLK // 128
GRP = 32
_TAIL_VALID = NUM_TYPES - (NT - 1) * T_BLK
_TAIL_LO = (_TAIL_VALID // 128) * 128

NEG = -3.0e38
BIGI = 2**30


@functools.lru_cache(maxsize=None)
def _make_sc_gather(V, D, N):
  info = plsc.get_sparse_core_info()
  NC, NS = info.num_cores, info.num_subcores
  NW = NC * NS
  assert D % 128 == 0 and N % (8 * NW) == 0
  n_per_w = N // NW
  mesh = plsc.VectorSubcoreMesh(core_axis_name="c", subcore_axis_name="s")

  @functools.partial(
      pl.kernel,
      mesh=mesh,
      out_type=jax.ShapeDtypeStruct((N, D), jnp.float32),
      scratch_types=[
          pltpu.VMEM((n_per_w,), jnp.int32),
          pltpu.VMEM((n_per_w, D), jnp.float32),
          pltpu.SemaphoreType.DMA,
      ],
  )
  def gather_k(table_hbm, idx_hbm, out_hbm, idx_v, rows_v, sem):
    wid = lax.axis_index("s") * NC + lax.axis_index("c")
    base = wid * n_per_w
    pltpu.sync_copy(idx_hbm.at[pl.ds(base, n_per_w)], idx_v)
    pltpu.async_copy(table_hbm.at[idx_v], rows_v, sem).wait()
    pltpu.sync_copy(rows_v, out_hbm.at[pl.ds(base, n_per_w)])

  return gather_k


def _prep_body(qf, nbf, qtr, pq, w_enc, b_enc, w_nb, b_nb,
               w1t, w1b, b1, w2, b2, wp_q, b_proj, cb_out, qp_out):
  q = jax.nn.relu(jnp.dot(qf[...], w_enc[...],
                          preferred_element_type=jnp.float32) + b_enc[...])
  acc = jax.nn.relu(jnp.dot(nbf[0], w_nb[...],
                            preferred_element_type=jnp.float32) + b_nb[...])
  for j in range(1, NUM_NEIGHBORS):
    acc = acc + jax.nn.relu(jnp.dot(nbf[j], w_nb[...],
                                    preferred_element_type=jnp.float32)
                            + b_nb[...])
  q_emb = q + acc * jnp.float32(1.0 / NUM_NEIGHBORS)
  qp_out[...] = jnp.dot(q_emb, wp_q[...],
                        preferred_element_type=jnp.float32) + b_proj[...]
  a = jnp.dot(qtr[...], w1t[...], preferred_element_type=jnp.float32)
  bb = jnp.dot(qtr[...], w1b[...], preferred_element_type=jnp.float32)
  h = jax.nn.relu(a + pq[...] * (bb - a) + b1[...])
  cb_out[...] = jnp.tanh(jnp.dot(h, w2[...],
                                 preferred_element_type=jnp.float32) + b2[...])


def _extract8(vals, idx):
  out_v, out_i = [], []
  for _ in range(K):
    m = jnp.max(vals, axis=1, keepdims=True)
    hit = vals == m
    gi = jnp.min(jnp.where(hit, idx, BIGI), axis=1, keepdims=True)
    out_v.append(m)
    out_i.append(gi)
    vals = jnp.where(idx == gi, NEG, vals)
  return jnp.concatenate(out_v, axis=1), jnp.concatenate(out_i, axis=1)


def _simtopk_body(cb_ref, tbl_ref, vals_out, idx_out, sims_s, sv, si):
  t = pl.program_id(0)

  @pl.when(t == 0)
  def _init():
    sv[...] = jnp.full(sv.shape, NEG, jnp.float32)
    si[...] = jnp.full(si.shape, BIGI, jnp.int32)

  sims_s[...] = lax.dot_general(cb_ref[...], tbl_ref[...],
                                (((1,), (1,)), ((), ())),
                                preferred_element_type=jnp.float32)

  @pl.when(t == NT - 1)
  def _mask_tail():
    reg = sims_s[:, _TAIL_LO:T_BLK]
    loc = lax.broadcasted_iota(jnp.int32, reg.shape, 1)
    sims_s[:, _TAIL_LO:T_BLK] = jnp.where(loc < _TAIL_VALID - _TAIL_LO,
                                          reg, NEG)

  iota = lax.broadcasted_iota(jnp.int32, (GRP, 128), 1)
  base_t = t * T_BLK

  def body(g):
    rows = pl.ds(g * GRP, GRP)
    Rv = [sv[rows, pl.ds(k * 128, 128)] for k in range(K)]
    Ri = [si[rows, pl.ds(k * 128, 128)] for k in range(K)]
    for s in range(NS_SLICES):
      x = sims_s[rows, pl.ds(s * 128, 128)]
      xi = iota + (base_t + s * 128)
      for k in range(K):
        c = x > Rv[k]
        nv = jnp.maximum(x, Rv[k])
        x = jnp.minimum(x, Rv[k])
        ni = jnp.where(c, xi, Ri[k])
        xi = jnp.where(c, Ri[k], xi)
        Rv[k] = nv
        Ri[k] = ni
    for k in range(K):
      sv[rows, pl.ds(k * 128, 128)] = Rv[k]
      si[rows, pl.ds(k * 128, 128)] = Ri[k]
    return g + 1

  lax.while_loop(lambda g: g < B // GRP, body, 0)

  @pl.when(t == NT - 1)
  def _emit():
    nv, ni = _extract8(sv[...], si[...])
    vals_out[...] = nv
    idx_out[...] = ni


def _proj_body(qp_ref, comp_ref, pk_ref, wpt_t, wpt_b, out_ref):
  a = jnp.dot(comp_ref[...], wpt_t[...], preferred_element_type=jnp.float32)
  bb = jnp.dot(comp_ref[...], wpt_b[...], preferred_element_type=jnp.float32)
  out_ref[...] = jnp.tanh(qp_ref[...] + a + pk_ref[...] * (bb - a))


def _pad_top_bot(w):
  z = jnp.zeros_like(w)
  return jnp.concatenate([w, z], axis=0), jnp.concatenate([z, w], axis=0)


def kernel(query_features, query_neighbor_features, query_types,
           W_enc, b_enc, W_nb, b_nb, W1, b1, W2, b2, W_proj, b_proj,
           query_type_table, comp_type_table):
  f32 = jnp.float32
  qtypes = query_types.astype(jnp.int32)

  qtable2 = query_type_table.reshape(NUM_TYPES // 2, 2 * TYPE_EMB_DIM)
  ctable2 = comp_type_table.reshape(NUM_TYPES // 2, 2 * TYPE_EMB_DIM)

  qt_rows = _make_sc_gather(NUM_TYPES // 2, 2 * TYPE_EMB_DIM, B)(
      qtable2, qtypes >> 1)
  p_q = (qtypes & 1).astype(f32).reshape(B, 1)

  nbf_t = query_neighbor_features.transpose(1, 0, 2)
  wp_q = W_proj[:PRODUCT_EMB_DIM]
  wp_t = W_proj[PRODUCT_EMB_DIM:]
  w1_t, w1_b = _pad_top_bot(W1)
  wpt_t, wpt_b = _pad_top_bot(wp_t)

  comp_base, qp = pl.pallas_call(
      _prep_body,
      out_shape=(jax.ShapeDtypeStruct((B, TYPE_EMB_DIM), f32),
                 jax.ShapeDtypeStruct((B, PRODUCT_EMB_DIM), f32)),
  )(query_features, nbf_t, qt_rows, p_q,
    W_enc, b_enc.reshape(1, -1), W_nb, b_nb.reshape(1, -1),
    w1_t, w1_b, b1.reshape(1, -1), W2, b2.reshape(1, -1),
    wp_q, b_proj.reshape(1, -1))

  topk_vals, topk_idx = pl.pallas_call(
      _simtopk_body,
      grid=(NT,),
      in_specs=[
          pl.BlockSpec((B, TYPE_EMB_DIM), lambda t: (0, 0)),
          pl.BlockSpec((T_BLK, TYPE_EMB_DIM), lambda t: (t, 0)),
      ],
      out_specs=[
          pl.BlockSpec((B, K), lambda t: (0, 0)),
          pl.BlockSpec((B, K), lambda t: (0, 0)),
      ],
      out_shape=(jax.ShapeDtypeStruct((B, K), f32),
                 jax.ShapeDtypeStruct((B, K), jnp.int32)),
      scratch_shapes=[
          pltpu.VMEM((B, T_BLK), f32),
          pltpu.VMEM((B, K * 128), f32),
          pltpu.VMEM((B, K * 128), jnp.int32),
      ],
  )(comp_base, comp_type_table)

  idx_km = topk_idx.T.reshape(B * K)
  comp_km = _make_sc_gather(NUM_TYPES // 2, 2 * TYPE_EMB_DIM, B * K)(
      ctable2, idx_km >> 1)
  p_km = (idx_km & 1).astype(f32).reshape(B * K, 1)

  out_km = pl.pallas_call(
      _proj_body,
      grid=(K,),
      in_specs=[
          pl.BlockSpec((B, PRODUCT_EMB_DIM), lambda k: (0, 0)),
          pl.BlockSpec((B, 2 * TYPE_EMB_DIM), lambda k: (k, 0)),
          pl.BlockSpec((B, 1), lambda k: (k, 0)),
          pl.BlockSpec((2 * TYPE_EMB_DIM, PRODUCT_EMB_DIM), lambda k: (0, 0)),
          pl.BlockSpec((2 * TYPE_EMB_DIM, PRODUCT_EMB_DIM), lambda k: (0, 0)),
      ],
      out_specs=pl.BlockSpec((B, PRODUCT_EMB_DIM), lambda k: (k, 0)),
      out_shape=jax.ShapeDtypeStruct((B * K, PRODUCT_EMB_DIM), f32),
  )(qp, comp_km, p_km, wpt_t, wpt_b)

  projected = out_km.reshape(K, B, PRODUCT_EMB_DIM).transpose(1, 0, 2)
  return projected, topk_idx, topk_vals


# speedup vs baseline: 1.2289x; 1.2289x over previous
import functools

import jax
import jax.numpy as jnp
from jax import lax
from jax.experimental import pallas as pl
from jax.experimental.pallas import tpu as pltpu
from jax.experimental.pallas import tpu_sc as plsc

B = 1024
NUM_TYPES = 100000
TYPE_EMB_DIM = 64
PRODUCT_EMB_DIM = 128
FEATURE_DIM = 128
HIDDEN_DIM = 128
K = 8
NUM_NEIGHBORS = 10

T_BLK = 2048
NT = (NUM_TYPES + T_BLK - 1) // T_BLK
NS_SLICES = T_BLK // 128
GRP = 16
_TAIL_VALID = NUM_TYPES - (NT - 1) * T_BLK
_TAIL_LO = (_TAIL_VALID // 128) * 128

NEG = -3.0e38
BIGI = 2**30


@functools.lru_cache(maxsize=None)
def _make_sc_gather(V, D, N):
  info = plsc.get_sparse_core_info()
  NC, NS = info.num_cores, info.num_subcores
  NW = NC * NS
  assert D % 128 == 0 and N % (8 * NW) == 0
  n_per_w = N // NW
  mesh = plsc.VectorSubcoreMesh(core_axis_name="c", subcore_axis_name="s")

  @functools.partial(
      pl.kernel,
      mesh=mesh,
      out_type=jax.ShapeDtypeStruct((N, D), jnp.float32),
      scratch_types=[
          pltpu.VMEM((n_per_w,), jnp.int32),
          pltpu.VMEM((n_per_w, D), jnp.float32),
          pltpu.SemaphoreType.DMA,
      ],
  )
  def gather_k(table_hbm, idx_hbm, out_hbm, idx_v, rows_v, sem):
    wid = lax.axis_index("s") * NC + lax.axis_index("c")
    base = wid * n_per_w
    pltpu.sync_copy(idx_hbm.at[pl.ds(base, n_per_w)], idx_v)
    pltpu.async_copy(table_hbm.at[idx_v], rows_v, sem).wait()
    pltpu.sync_copy(rows_v, out_hbm.at[pl.ds(base, n_per_w)])

  return gather_k


def _prep_body(qf, nbf, qtr, pq, w_enc, b_enc, w_nb, b_nb,
               w1t, w1b, b1, w2, b2, wp_q, b_proj, cb_out, qp_out):
  q = jax.nn.relu(jnp.dot(qf[...], w_enc[...],
                          preferred_element_type=jnp.float32) + b_enc[...])
  acc = jax.nn.relu(jnp.dot(nbf[0], w_nb[...],
                            preferred_element_type=jnp.float32) + b_nb[...])
  for j in range(1, NUM_NEIGHBORS):
    acc = acc + jax.nn.relu(jnp.dot(nbf[j], w_nb[...],
                                    preferred_element_type=jnp.float32)
                            + b_nb[...])
  q_emb = q + acc * jnp.float32(1.0 / NUM_NEIGHBORS)
  qp_out[...] = jnp.dot(q_emb, wp_q[...],
                        preferred_element_type=jnp.float32) + b_proj[...]
  a = jnp.dot(qtr[...], w1t[...], preferred_element_type=jnp.float32)
  bb = jnp.dot(qtr[...], w1b[...], preferred_element_type=jnp.float32)
  h = jax.nn.relu(a + pq[...] * (bb - a) + b1[...])
  cb_out[...] = jnp.tanh(jnp.dot(h, w2[...],
                                 preferred_element_type=jnp.float32) + b2[...])


def _extract8(vals, idx):
  out_v, out_i = [], []
  for _ in range(K):
    m = jnp.max(vals, axis=1, keepdims=True)
    hit = vals == m
    gi = jnp.min(jnp.where(hit, idx, BIGI), axis=1, keepdims=True)
    out_v.append(m)
    out_i.append(gi)
    vals = jnp.where(idx == gi, NEG, vals)
  return jnp.concatenate(out_v, axis=1), jnp.concatenate(out_i, axis=1)


def _simtopk_body(cb_ref, tbl_ref, vals_out, idx_out, sims_s, sv, si):
  t = pl.program_id(0)

  @pl.when(t == 0)
  def _init():
    sv[...] = jnp.full(sv.shape, NEG, jnp.float32)
    si[...] = jnp.full(si.shape, BIGI, jnp.int32)

  sims_s[...] = lax.dot_general(cb_ref[...], tbl_ref[...],
                                (((1,), (1,)), ((), ())),
                                preferred_element_type=jnp.float32)

  @pl.when(t == NT - 1)
  def _mask_tail():
    reg = sims_s[:, _TAIL_LO:T_BLK]
    loc = lax.broadcasted_iota(jnp.int32, reg.shape, 1)
    sims_s[:, _TAIL_LO:T_BLK] = jnp.where(loc < _TAIL_VALID - _TAIL_LO,
                                          reg, NEG)

  iota = lax.broadcasted_iota(jnp.int32, (GRP, 128), 1)
  base_t = t * T_BLK

  def body(g):
    rows = pl.ds(g * GRP, GRP)
    Rv = [sv[rows, pl.ds(k * 128, 128)] for k in range(K)]
    Ri = [si[rows, pl.ds(k * 128, 128)] for k in range(K)]
    for s in range(NS_SLICES):
      x = sims_s[rows, pl.ds(s * 128, 128)]
      xi = iota + (base_t + s * 128)
      for k in range(K):
        c = x > Rv[k]
        nv = jnp.maximum(x, Rv[k])
        x = jnp.minimum(x, Rv[k])
        ni = jnp.where(c, xi, Ri[k])
        xi = jnp.where(c, Ri[k], xi)
        Rv[k] = nv
        Ri[k] = ni
    for k in range(K):
      sv[rows, pl.ds(k * 128, 128)] = Rv[k]
      si[rows, pl.ds(k * 128, 128)] = Ri[k]
    return g + 1

  lax.while_loop(lambda g: g < B // GRP, body, 0)

  @pl.when(t == NT - 1)
  def _emit():
    nv, ni = _extract8(sv[...], si[...])
    vals_out[...] = nv
    idx_out[...] = ni


def _proj_body(qp_ref, comp_ref, pk_ref, wpt_t, wpt_b, out_ref):
  a = jnp.dot(comp_ref[...], wpt_t[...], preferred_element_type=jnp.float32)
  bb = jnp.dot(comp_ref[...], wpt_b[...], preferred_element_type=jnp.float32)
  out_ref[...] = jnp.tanh(qp_ref[...] + a + pk_ref[...] * (bb - a))


def _pad_top_bot(w):
  z = jnp.zeros_like(w)
  return jnp.concatenate([w, z], axis=0), jnp.concatenate([z, w], axis=0)


def kernel(query_features, query_neighbor_features, query_types,
           W_enc, b_enc, W_nb, b_nb, W1, b1, W2, b2, W_proj, b_proj,
           query_type_table, comp_type_table):
  f32 = jnp.float32
  qtypes = query_types.astype(jnp.int32)

  qtable2 = query_type_table.reshape(NUM_TYPES // 2, 2 * TYPE_EMB_DIM)
  ctable2 = comp_type_table.reshape(NUM_TYPES // 2, 2 * TYPE_EMB_DIM)

  qt_rows = _make_sc_gather(NUM_TYPES // 2, 2 * TYPE_EMB_DIM, B)(
      qtable2, qtypes >> 1)
  p_q = (qtypes & 1).astype(f32).reshape(B, 1)

  nbf_t = query_neighbor_features.transpose(1, 0, 2)
  wp_q = W_proj[:PRODUCT_EMB_DIM]
  wp_t = W_proj[PRODUCT_EMB_DIM:]
  w1_t, w1_b = _pad_top_bot(W1)
  wpt_t, wpt_b = _pad_top_bot(wp_t)

  comp_base, qp = pl.pallas_call(
      _prep_body,
      out_shape=(jax.ShapeDtypeStruct((B, TYPE_EMB_DIM), f32),
                 jax.ShapeDtypeStruct((B, PRODUCT_EMB_DIM), f32)),
  )(query_features, nbf_t, qt_rows, p_q,
    W_enc, b_enc.reshape(1, -1), W_nb, b_nb.reshape(1, -1),
    w1_t, w1_b, b1.reshape(1, -1), W2, b2.reshape(1, -1),
    wp_q, b_proj.reshape(1, -1))

  topk_vals, topk_idx = pl.pallas_call(
      _simtopk_body,
      grid=(NT,),
      in_specs=[
          pl.BlockSpec((B, TYPE_EMB_DIM), lambda t: (0, 0)),
          pl.BlockSpec((T_BLK, TYPE_EMB_DIM), lambda t: (t, 0)),
      ],
      out_specs=[
          pl.BlockSpec((B, K), lambda t: (0, 0)),
          pl.BlockSpec((B, K), lambda t: (0, 0)),
      ],
      out_shape=(jax.ShapeDtypeStruct((B, K), f32),
                 jax.ShapeDtypeStruct((B, K), jnp.int32)),
      scratch_shapes=[
          pltpu.VMEM((B, T_BLK), f32),
          pltpu.VMEM((B, K * 128), f32),
          pltpu.VMEM((B, K * 128), jnp.int32),
      ],
  )(comp_base, comp_type_table)

  idx_km = topk_idx.T.reshape(B * K)
  comp_km = _make_sc_gather(NUM_TYPES // 2, 2 * TYPE_EMB_DIM, B * K)(
      ctable2, idx_km >> 1)
  p_km = (idx_km & 1).astype(f32).reshape(B * K, 1)

  out_km = pl.pallas_call(
      _proj_body,
      grid=(K,),
      in_specs=[
          pl.BlockSpec((B, PRODUCT_EMB_DIM), lambda k: (0, 0)),
          pl.BlockSpec((B, 2 * TYPE_EMB_DIM), lambda k: (k, 0)),
          pl.BlockSpec((B, 1), lambda k: (k, 0)),
          pl.BlockSpec((2 * TYPE_EMB_DIM, PRODUCT_EMB_DIM), lambda k: (0, 0)),
          pl.BlockSpec((2 * TYPE_EMB_DIM, PRODUCT_EMB_DIM), lambda k: (0, 0)),
      ],
      out_specs=pl.BlockSpec((B, PRODUCT_EMB_DIM), lambda k: (k, 0)),
      out_shape=jax.ShapeDtypeStruct((B * K, PRODUCT_EMB_DIM), f32),
  )(qp, comp_km, p_km, wpt_t, wpt_b)

  projected = out_km.reshape(K, B, PRODUCT_EMB_DIM).transpose(1, 0, 2)
  return projected, topk_idx, topk_vals
